# baseline (speedup 1.0000x reference)
import jax
import jax.numpy as jnp
from jax.experimental import pallas as pl
from jax.experimental.pallas import tpu as pltpu


def _router_kernel(x_ref, rs_ref, wr_ref, pes_ref, cw_ref):
    xb = x_ref[...]
    d = xb.shape[-1]
    ms = jnp.mean(jnp.square(xb), axis=-1, keepdims=True)
    rn = xb * jax.lax.rsqrt(ms + 1e-6)
    ri = rn * jax.lax.rsqrt(jnp.float32(d)) * rs_ref[...]
    logits = jnp.dot(ri, wr_ref[...], preferred_element_type=jnp.float32)
    probs = jax.nn.softmax(logits, axis=-1)
    m1 = jnp.max(logits, axis=-1, keepdims=True)
    masked = jnp.where(logits >= m1, -jnp.inf, logits)
    m2 = jnp.max(masked, axis=-1, keepdims=True)
    mask = (logits >= m2).astype(jnp.float32)
    gw = probs * mask
    rf = jnp.sum(gw, axis=-1, keepdims=True)
    rf = jnp.where(rf > 0.0, rf, 1.0)
    cw_ref[...] = gw / rf * pes_ref[...]


def _moe_kernel(cw_ref, x_ref, w0_ref, w1_ref, wl_ref, out_ref):
    e = pl.program_id(1)
    fb = pl.program_id(2)
    xb = x_ref[...]
    w0 = w0_ref[0]
    w1 = w1_ref[0]
    wl = wl_ref[0]
    dn = (((1,), (1,)), ((), ()))
    h1 = jax.lax.dot_general(xb, w0, dn, preferred_element_type=jnp.float32)
    h2 = jax.lax.dot_general(xb, w1, dn, preferred_element_type=jnp.float32)
    h = (jax.nn.gelu(h1) * h2).astype(jnp.bfloat16)
    y = jax.lax.dot_general(h, wl, (((1,), (0,)), ((), ())),
                            preferred_element_type=jnp.float32)
    cwb = cw_ref[...]
    n_e = cwb.shape[-1]
    oh = (jax.lax.broadcasted_iota(jnp.int32, (n_e, 1), 0) == e).astype(
        jnp.float32)
    col = jnp.dot(cwb, oh, preferred_element_type=jnp.float32)
    y = y * col

    @pl.when((e == 0) & (fb == 0))
    def _init():
        out_ref[...] = y

    @pl.when(jnp.logical_not((e == 0) & (fb == 0)))
    def _acc():
        out_ref[...] += y


def kernel(x, router_scale, per_expert_scale, w_router, w_gating, w_linear):
    g, s, d = x.shape
    e_num, _, f, _ = w_gating.shape
    sb = 2048
    fbk = 512
    nsb = s // sb
    nfb = f // fbk

    xf = x.reshape(s, d).astype(jnp.float32)
    cw = pl.pallas_call(
        _router_kernel,
        out_shape=jax.ShapeDtypeStruct((s, e_num), jnp.float32),
    )(xf, router_scale.reshape(1, d).astype(jnp.float32),
      w_router.astype(jnp.float32),
      per_expert_scale.reshape(1, e_num).astype(jnp.float32))

    wg16 = w_gating.astype(jnp.bfloat16)
    w0 = wg16[:, 0]
    w1 = wg16[:, 1]
    wl16 = w_linear.astype(jnp.bfloat16)
    x16 = xf.astype(jnp.bfloat16)

    out = pl.pallas_call(
        _moe_kernel,
        grid=(nsb, e_num, nfb),
        in_specs=[
            pl.BlockSpec((sb, e_num), lambda i, e, j: (i, 0)),
            pl.BlockSpec((sb, d), lambda i, e, j: (i, 0)),
            pl.BlockSpec((1, fbk, d), lambda i, e, j: (e, j, 0)),
            pl.BlockSpec((1, fbk, d), lambda i, e, j: (e, j, 0)),
            pl.BlockSpec((1, fbk, d), lambda i, e, j: (e, j, 0)),
        ],
        out_specs=pl.BlockSpec((sb, d), lambda i, e, j: (i, 0)),
        out_shape=jax.ShapeDtypeStruct((s, d), jnp.float32),
        compiler_params=pltpu.CompilerParams(
            dimension_semantics=("parallel", "arbitrary", "arbitrary"),
        ),
    )(cw, x16, w0, w1, wl16)
    return out.reshape(g, s, d)


# speedup vs baseline: 1.1331x; 1.1331x over previous
import jax
import jax.numpy as jnp
from jax.experimental import pallas as pl
from jax.experimental.pallas import tpu as pltpu


def _router_kernel(x_ref, rs_ref, wr_ref, pes_ref, cw_ref):
    xb = x_ref[...]
    d = xb.shape[-1]
    ms = jnp.mean(jnp.square(xb), axis=-1, keepdims=True)
    rn = xb * jax.lax.rsqrt(ms + 1e-6)
    ri = rn * jax.lax.rsqrt(jnp.float32(d)) * rs_ref[...]
    logits = jnp.dot(ri, wr_ref[...], preferred_element_type=jnp.float32)
    probs = jax.nn.softmax(logits, axis=-1)
    m1 = jnp.max(logits, axis=-1, keepdims=True)
    masked = jnp.where(logits >= m1, -jnp.inf, logits)
    m2 = jnp.max(masked, axis=-1, keepdims=True)
    mask = (logits >= m2).astype(jnp.float32)
    gw = probs * mask
    rf = jnp.sum(gw, axis=-1, keepdims=True)
    rf = jnp.where(rf > 0.0, rf, 1.0)
    cw_ref[...] = gw / rf * pes_ref[...]


def _moe_kernel(cw_ref, x_ref, w0_ref, w1_ref, wl_ref, out_ref):
    e = pl.program_id(1)
    fb = pl.program_id(2)
    xb = x_ref[...]
    w0 = w0_ref[0]
    w1 = w1_ref[0]
    wl = wl_ref[0]
    dn = (((1,), (1,)), ((), ()))
    h1 = jax.lax.dot_general(xb, w0, dn, preferred_element_type=jnp.float32)
    h2 = jax.lax.dot_general(xb, w1, dn, preferred_element_type=jnp.float32)
    h = (jax.nn.gelu(h1) * h2).astype(jnp.bfloat16)
    y = jax.lax.dot_general(h, wl, (((1,), (0,)), ((), ())),
                            preferred_element_type=jnp.float32)
    cwb = cw_ref[...]
    n_e = cwb.shape[-1]
    oh = (jax.lax.broadcasted_iota(jnp.int32, (n_e, 1), 0) == e).astype(
        jnp.float32)
    col = jnp.dot(cwb, oh, preferred_element_type=jnp.float32)
    y = y * col

    @pl.when((e == 0) & (fb == 0))
    def _init():
        out_ref[...] = y

    @pl.when(jnp.logical_not((e == 0) & (fb == 0)))
    def _acc():
        out_ref[...] += y


def kernel(x, router_scale, per_expert_scale, w_router, w_gating, w_linear):
    g, s, d = x.shape
    e_num, _, f, _ = w_gating.shape
    sb = 1024
    fbk = 2048
    nsb = s // sb
    nfb = f // fbk

    xf = x.reshape(s, d).astype(jnp.float32)
    cw = pl.pallas_call(
        _router_kernel,
        out_shape=jax.ShapeDtypeStruct((s, e_num), jnp.float32),
    )(xf, router_scale.reshape(1, d).astype(jnp.float32),
      w_router.astype(jnp.float32),
      per_expert_scale.reshape(1, e_num).astype(jnp.float32))

    wg16 = w_gating.astype(jnp.bfloat16)
    w0 = wg16[:, 0]
    w1 = wg16[:, 1]
    wl16 = w_linear.astype(jnp.bfloat16)
    x16 = xf.astype(jnp.bfloat16)

    out = pl.pallas_call(
        _moe_kernel,
        grid=(nsb, e_num, nfb),
        in_specs=[
            pl.BlockSpec((sb, e_num), lambda i, e, j: (i, 0)),
            pl.BlockSpec((sb, d), lambda i, e, j: (i, 0)),
            pl.BlockSpec((1, fbk, d), lambda i, e, j: (e, j, 0)),
            pl.BlockSpec((1, fbk, d), lambda i, e, j: (e, j, 0)),
            pl.BlockSpec((1, fbk, d), lambda i, e, j: (e, j, 0)),
        ],
        out_specs=pl.BlockSpec((sb, d), lambda i, e, j: (i, 0)),
        out_shape=jax.ShapeDtypeStruct((s, d), jnp.float32),
        compiler_params=pltpu.CompilerParams(
            dimension_semantics=("parallel", "arbitrary", "arbitrary"),
        ),
    )(cw, x16, w0, w1, wl16)
    return out.reshape(g, s, d)
